# Initial kernel scaffold; baseline (speedup 1.0000x reference)
import functools

import jax
import jax.numpy as jnp
from jax import lax
from jax.experimental import pallas as pl
from jax.experimental.pallas import tpu as pltpu
from jax.experimental.pallas import tpu_sc as plsc

N = 10000
E = 320000
D = 128

NC = 2
NS = 16
NW = NC * NS

EDGES_PER_W = E // NW
CHUNK = 80
NCHUNK = EDGES_PER_W // CHUNK

ROWS_PER_TILE = N // NS
DEGW_PAD = 10240
DEGW_PER_TILE = DEGW_PAD // NS


def _sc_body(x_hbm, src_hbm, dst_hbm, w_hbm, agg_hbm, degw_hbm,
             agg_sh, degw_sh, src_v, dst_v, w_v, rows, dz):
    c = lax.axis_index("c")
    s = lax.axis_index("s")
    wid = s * NC + c
    base = wid * EDGES_PER_W

    zeros16 = jnp.zeros((16,), jnp.float32)

    def _zero_rows(r, _):
        for g in range(8):
            rows[r, pl.ds(g * 16, 16)] = zeros16
        return 0

    lax.fori_loop(0, CHUNK, _zero_rows, 0)

    def _zero_dz(i, _):
        dz[pl.ds(i * 16, 16)] = zeros16
        return 0

    lax.fori_loop(0, DEGW_PER_TILE // 16, _zero_dz, 0)

    row0 = s * ROWS_PER_TILE
    for k in range(7):
        pltpu.sync_copy(rows.at[pl.ds(0, CHUNK)],
                        agg_sh.at[pl.ds(row0 + k * CHUNK, CHUNK)])
    pltpu.sync_copy(rows.at[pl.ds(0, ROWS_PER_TILE - 7 * CHUNK)],
                    agg_sh.at[pl.ds(row0 + 7 * CHUNK, ROWS_PER_TILE - 7 * CHUNK)])
    pltpu.sync_copy(dz, degw_sh.at[pl.ds(s * DEGW_PER_TILE, DEGW_PER_TILE)])
    plsc.subcore_barrier()

    def _chunk(j, _):
        off = base + j * CHUNK
        pltpu.sync_copy(src_hbm.at[pl.ds(off, CHUNK)], src_v)
        pltpu.sync_copy(dst_hbm.at[pl.ds(off, CHUNK)], dst_v)
        pltpu.sync_copy(w_hbm.at[pl.ds(off, CHUNK)], w_v)
        pltpu.sync_copy(x_hbm.at[src_v], rows)

        def _scale(e, _):
            wb = plsc.load_gather(w_v, [jnp.full((16,), e, jnp.int32)])
            for g in range(8):
                sl = pl.ds(g * 16, 16)
                rows[e, sl] = rows[e, sl] * wb
            return 0

        lax.fori_loop(0, CHUNK, _scale, 0)

        pltpu.sync_copy(rows, agg_sh.at[dst_v], add=True)
        pltpu.sync_copy(w_v, degw_sh.at[dst_v], add=True)
        return 0

    lax.fori_loop(0, NCHUNK, _chunk, 0)
    plsc.subcore_barrier()

    out_row0 = c * N + s * ROWS_PER_TILE
    pltpu.sync_copy(agg_sh.at[pl.ds(s * ROWS_PER_TILE, ROWS_PER_TILE)],
                    agg_hbm.at[pl.ds(out_row0, ROWS_PER_TILE)])
    pltpu.sync_copy(degw_sh.at[pl.ds(s * DEGW_PER_TILE, DEGW_PER_TILE)],
                    degw_hbm.at[pl.ds(c * DEGW_PAD + s * DEGW_PER_TILE,
                                      DEGW_PER_TILE)])


_sc_call = functools.partial(
    pl.kernel,
    out_type=(jax.ShapeDtypeStruct((NC * N, D), jnp.float32),
              jax.ShapeDtypeStruct((NC * DEGW_PAD,), jnp.float32)),
    mesh=plsc.VectorSubcoreMesh(core_axis_name="c", subcore_axis_name="s"),
    scratch_types=(
        pltpu.VMEM_SHARED((N, D), jnp.float32),
        pltpu.VMEM_SHARED((DEGW_PAD,), jnp.float32),
        pltpu.VMEM((CHUNK,), jnp.int32),
        pltpu.VMEM((CHUNK,), jnp.int32),
        pltpu.VMEM((CHUNK,), jnp.float32),
        pltpu.VMEM((CHUNK, D), jnp.float32),
        pltpu.VMEM((DEGW_PER_TILE,), jnp.float32),
    ),
)(_sc_body)


def _tc_body(p_ref, degw_ref, w_ref, b_ref, out_ref):
    agg = p_ref[0] + p_ref[1]
    dw = degw_ref[..., 0] + degw_ref[..., 1]
    out_ref[...] = (
        lax.dot_general(agg, w_ref[...], (((1,), (1,)), ((), ())),
                        preferred_element_type=jnp.float32)
        + dw[:, None] * b_ref[...]
    )


TC_BLOCK = 1000


def _tc_call(p, degw2, W_lin, b2):
    return pl.pallas_call(
        _tc_body,
        grid=(N // TC_BLOCK,),
        in_specs=[
            pl.BlockSpec((NC, TC_BLOCK, D), lambda i: (0, i, 0)),
            pl.BlockSpec((TC_BLOCK, NC), lambda i: (i, 0)),
            pl.BlockSpec((D, D), lambda i: (0, 0)),
            pl.BlockSpec((1, D), lambda i: (0, 0)),
        ],
        out_specs=pl.BlockSpec((TC_BLOCK, D), lambda i: (i, 0)),
        out_shape=jax.ShapeDtypeStruct((N, D), jnp.float32),
    )(p, degw2, W_lin, b2)


@jax.jit
def kernel(x, edge_index, edge_weight, W_lin, b_lin):
    src = edge_index[1]
    dst = edge_index[0]
    agg_flat, degw_flat = _sc_call(x, src, dst, edge_weight)
    p = agg_flat.reshape(NC, N, D)
    degw2 = jnp.stack(
        [degw_flat[:N], degw_flat[DEGW_PAD:DEGW_PAD + N]], axis=1)
    return _tc_call(p, degw2, W_lin, b_lin.reshape(1, D))


# speedup vs baseline: 3.9277x; 3.9277x over previous
import functools

import jax
import jax.numpy as jnp
from jax import lax
from jax.experimental import pallas as pl
from jax.experimental.pallas import tpu as pltpu
from jax.experimental.pallas import tpu_sc as plsc

N = 10000
E = 320000
D = 128

NC = 2
NS = 16
NW = NC * NS

EDGES_PER_W = E // NW
CHUNK = 80
NCHUNK = EDGES_PER_W // CHUNK

ROWS_PER_TILE = 624
ROWS_TAIL = N - NS * ROWS_PER_TILE
DEGW_PAD = 10240
DEGW_PER_TILE = DEGW_PAD // NS


def _sc_body(x_hbm, src_hbm, dst_hbm, w_hbm, agg_hbm, degw_hbm,
             agg_sh, degw_sh, src_v, dst_v, w_v, rows, dz):
    c = lax.axis_index("c")
    s = lax.axis_index("s")
    wid = s * NC + c
    base = wid * EDGES_PER_W

    zeros16 = jnp.zeros((16,), jnp.float32)

    def _zero_rows(r, _):
        for g in range(8):
            rows[r, pl.ds(g * 16, 16)] = zeros16
        return 0

    lax.fori_loop(0, CHUNK, _zero_rows, 0)

    def _zero_dz(i, _):
        dz[pl.ds(i * 16, 16)] = zeros16
        return 0

    lax.fori_loop(0, DEGW_PER_TILE // 16, _zero_dz, 0)

    row0 = s * ROWS_PER_TILE
    for k in range(7):
        pltpu.sync_copy(rows.at[pl.ds(0, CHUNK)],
                        agg_sh.at[pl.ds(row0 + k * CHUNK, CHUNK)])
    pltpu.sync_copy(rows.at[pl.ds(0, ROWS_PER_TILE - 7 * CHUNK)],
                    agg_sh.at[pl.ds(row0 + 7 * CHUNK, ROWS_PER_TILE - 7 * CHUNK)])

    @pl.when(s == NS - 1)
    def _zero_tail():
        pltpu.sync_copy(rows.at[pl.ds(0, ROWS_TAIL)],
                        agg_sh.at[pl.ds(NS * ROWS_PER_TILE, ROWS_TAIL)])
    pltpu.sync_copy(dz, degw_sh.at[pl.ds(s * DEGW_PER_TILE, DEGW_PER_TILE)])
    plsc.subcore_barrier()

    def _chunk(j, _):
        off = base + j * CHUNK
        pltpu.sync_copy(src_hbm.at[pl.ds(off, CHUNK)], src_v)
        pltpu.sync_copy(dst_hbm.at[pl.ds(off, CHUNK)], dst_v)
        pltpu.sync_copy(w_hbm.at[pl.ds(off, CHUNK)], w_v)
        pltpu.sync_copy(x_hbm.at[src_v], rows)

        def _scale(e, _):
            wb = plsc.load_gather(w_v, [jnp.full((16,), e, jnp.int32)])
            for g in range(8):
                sl = pl.ds(g * 16, 16)
                rows[e, sl] = rows[e, sl] * wb
            return 0

        lax.fori_loop(0, CHUNK, _scale, 0)

        pltpu.sync_copy(rows, agg_sh.at[dst_v], add=True)
        pltpu.sync_copy(w_v, degw_sh.at[dst_v], add=True)
        return 0

    lax.fori_loop(0, NCHUNK, _chunk, 0)
    plsc.subcore_barrier()

    out_row0 = c * N + s * ROWS_PER_TILE
    pltpu.sync_copy(agg_sh.at[pl.ds(s * ROWS_PER_TILE, ROWS_PER_TILE)],
                    agg_hbm.at[pl.ds(out_row0, ROWS_PER_TILE)])

    @pl.when(s == NS - 1)
    def _drain_tail():
        pltpu.sync_copy(agg_sh.at[pl.ds(NS * ROWS_PER_TILE, ROWS_TAIL)],
                        agg_hbm.at[pl.ds(c * N + NS * ROWS_PER_TILE,
                                         ROWS_TAIL)])
    pltpu.sync_copy(degw_sh.at[pl.ds(s * DEGW_PER_TILE, DEGW_PER_TILE)],
                    degw_hbm.at[pl.ds(c * DEGW_PAD + s * DEGW_PER_TILE,
                                      DEGW_PER_TILE)])


_sc_call = functools.partial(
    pl.kernel,
    out_type=(jax.ShapeDtypeStruct((NC * N, D), jnp.float32),
              jax.ShapeDtypeStruct((NC * DEGW_PAD,), jnp.float32)),
    mesh=plsc.VectorSubcoreMesh(core_axis_name="c", subcore_axis_name="s"),
    compiler_params=pltpu.CompilerParams(needs_layout_passes=False),
    scratch_types=(
        pltpu.VMEM_SHARED((N, D), jnp.float32),
        pltpu.VMEM_SHARED((DEGW_PAD,), jnp.float32),
        pltpu.VMEM((CHUNK,), jnp.int32),
        pltpu.VMEM((CHUNK,), jnp.int32),
        pltpu.VMEM((CHUNK,), jnp.float32),
        pltpu.VMEM((CHUNK, D), jnp.float32),
        pltpu.VMEM((DEGW_PER_TILE,), jnp.float32),
    ),
)(_sc_body)


def _tc_body(p_ref, degw_ref, w_ref, b_ref, out_ref):
    agg = p_ref[0] + p_ref[1]
    dw = degw_ref[..., 0] + degw_ref[..., 1]
    out_ref[...] = (
        lax.dot_general(agg, w_ref[...], (((1,), (1,)), ((), ())),
                        preferred_element_type=jnp.float32)
        + dw[:, None] * b_ref[...]
    )


TC_BLOCK = 1000


def _tc_call(p, degw2, W_lin, b2):
    return pl.pallas_call(
        _tc_body,
        grid=(N // TC_BLOCK,),
        in_specs=[
            pl.BlockSpec((NC, TC_BLOCK, D), lambda i: (0, i, 0)),
            pl.BlockSpec((TC_BLOCK, NC), lambda i: (i, 0)),
            pl.BlockSpec((D, D), lambda i: (0, 0)),
            pl.BlockSpec((1, D), lambda i: (0, 0)),
        ],
        out_specs=pl.BlockSpec((TC_BLOCK, D), lambda i: (i, 0)),
        out_shape=jax.ShapeDtypeStruct((N, D), jnp.float32),
    )(p, degw2, W_lin, b2)


@jax.jit
def kernel(x, edge_index, edge_weight, W_lin, b_lin):
    src = edge_index[1]
    dst = edge_index[0]
    agg_flat, degw_flat = _sc_call(x, src, dst, edge_weight)
    p = agg_flat.reshape(NC, N, D)
    degw2 = jnp.stack(
        [degw_flat[:N], degw_flat[DEGW_PAD:DEGW_PAD + N]], axis=1)
    return _tc_call(p, degw2, W_lin, b_lin.reshape(1, D))


# speedup vs baseline: 5.9640x; 1.5184x over previous
import functools

import jax
import jax.numpy as jnp
from jax import lax
from jax.experimental import pallas as pl
from jax.experimental.pallas import tpu as pltpu
from jax.experimental.pallas import tpu_sc as plsc

N = 10000
E = 320000
D = 128
DH = D // 2

NC = 2
NS = 16
CHUNK = 128
E_PAD = 2560 * CHUNK
CH_PER_TILE = E_PAD // NS // CHUNK
SB = 16
NSB = CH_PER_TILE // SB
NB = 4
UNROLL = 4

ROWS_PER_TILE = 624
ROWS_TAIL = N - NS * ROWS_PER_TILE
DEGW_PAD = 10240
DEGW_PER_TILE = DEGW_PAD // NS


def _sc_body(xs_hbm, src_hbm, dst_hbm, w_hbm, agg_hbm, degw_hbm,
             agg_sh, degw_sh, src_sb, dst_sb, w_sb, dz,
             rows0, rows1, rows2, rows3, gsem, ssem, dsem):
    c = lax.axis_index("c")
    s = lax.axis_index("s")
    rows = (rows0, rows1, rows2, rows3)

    zeros16 = jnp.zeros((16,), jnp.float32)

    def _zero_rows(r, _):
        for g in range(DH // 16):
            rows0[r, pl.ds(g * 16, 16)] = zeros16
        return 0

    lax.fori_loop(0, CHUNK, _zero_rows, 0)

    def _zero_dz(i, _):
        dz[pl.ds(i * 16, 16)] = zeros16
        return 0

    lax.fori_loop(0, DEGW_PER_TILE // 16, _zero_dz, 0)

    row0 = s * ROWS_PER_TILE
    for k in range(4):
        pltpu.sync_copy(rows0.at[pl.ds(0, CHUNK)],
                        agg_sh.at[pl.ds(row0 + k * CHUNK, CHUNK)])
    pltpu.sync_copy(rows0.at[pl.ds(0, ROWS_PER_TILE - 4 * CHUNK)],
                    agg_sh.at[pl.ds(row0 + 4 * CHUNK,
                                    ROWS_PER_TILE - 4 * CHUNK)])
    pltpu.sync_copy(dz, degw_sh.at[pl.ds(s * DEGW_PER_TILE, DEGW_PER_TILE)])

    @pl.when(s == NS - 1)
    def _zero_tail():
        pltpu.sync_copy(rows0.at[pl.ds(0, ROWS_TAIL)],
                        agg_sh.at[pl.ds(NS * ROWS_PER_TILE, ROWS_TAIL)])

    plsc.subcore_barrier()

    src_row0 = c * (E_PAD // CHUNK) + s * CH_PER_TILE
    ed_row0 = s * CH_PER_TILE

    def _scale_chunk(q, buf):
        def _scale(e4, _):
            for k in range(UNROLL):
                e = e4 * UNROLL + k
                ev = jnp.full((16,), e, jnp.int32)
                qv = jnp.full((16,), q, jnp.int32)
                wb = plsc.load_gather(w_sb, [qv, ev])
                for g in range(DH // 16):
                    sl = pl.ds(g * 16, 16)
                    buf[e, sl] = buf[e, sl] * wb
            return 0

        lax.fori_loop(0, CHUNK // UNROLL, _scale, 0)

    def _superblock(sb, _):
        pltpu.sync_copy(src_hbm.at[pl.ds(src_row0 + sb * SB, SB)], src_sb)
        pltpu.sync_copy(dst_hbm.at[pl.ds(ed_row0 + sb * SB, SB)], dst_sb)
        pltpu.sync_copy(w_hbm.at[pl.ds(ed_row0 + sb * SB, SB)], w_sb)

        def _group(qq, _):
            q0 = qq * NB
            gd = []
            for b in range(NB):
                gd.append(pltpu.async_copy(
                    xs_hbm.at[src_sb.at[q0 + b]], rows[b], gsem.at[b]))
            sd = []
            for b in range(NB):
                q = q0 + b
                gd[b].wait()
                _scale_chunk(q, rows[b])
                sd.append(pltpu.async_copy(
                    rows[b], agg_sh.at[dst_sb.at[q]], ssem.at[b], add=True))
                sd.append(pltpu.async_copy(
                    w_sb.at[q], degw_sh.at[dst_sb.at[q]], dsem.at[b],
                    add=True))
            for d in sd:
                d.wait()
            return 0

        lax.fori_loop(0, SB // NB, _group, 0)
        return 0

    lax.fori_loop(0, NSB, _superblock, 0)
    plsc.subcore_barrier()

    out_row0 = c * N + s * ROWS_PER_TILE
    pltpu.sync_copy(agg_sh.at[pl.ds(s * ROWS_PER_TILE, ROWS_PER_TILE)],
                    agg_hbm.at[pl.ds(out_row0, ROWS_PER_TILE)])

    @pl.when(s == NS - 1)
    def _drain_tail():
        pltpu.sync_copy(agg_sh.at[pl.ds(NS * ROWS_PER_TILE, ROWS_TAIL)],
                        agg_hbm.at[pl.ds(c * N + NS * ROWS_PER_TILE,
                                         ROWS_TAIL)])

    @pl.when(c == 0)
    def _drain_degw():
        pltpu.sync_copy(degw_sh.at[pl.ds(s * DEGW_PER_TILE, DEGW_PER_TILE)],
                        degw_hbm.at[pl.ds(s * DEGW_PER_TILE, DEGW_PER_TILE)])


_sc_call = functools.partial(
    pl.kernel,
    out_type=(jax.ShapeDtypeStruct((NC * N, DH), jnp.float32),
              jax.ShapeDtypeStruct((DEGW_PAD,), jnp.float32)),
    mesh=plsc.VectorSubcoreMesh(core_axis_name="c", subcore_axis_name="s"),
    compiler_params=pltpu.CompilerParams(needs_layout_passes=False, use_tc_tiling_on_sc=False),
    scratch_types=(
        pltpu.VMEM_SHARED((N, DH), jnp.float32),
        pltpu.VMEM_SHARED((DEGW_PAD,), jnp.float32),
        pltpu.VMEM((SB, CHUNK), jnp.int32),
        pltpu.VMEM((SB, CHUNK), jnp.int32),
        pltpu.VMEM((SB, CHUNK), jnp.float32),
        pltpu.VMEM((DEGW_PER_TILE,), jnp.float32),
        pltpu.VMEM((CHUNK, DH), jnp.float32),
        pltpu.VMEM((CHUNK, DH), jnp.float32),
        pltpu.VMEM((CHUNK, DH), jnp.float32),
        pltpu.VMEM((CHUNK, DH), jnp.float32),
        pltpu.SemaphoreType.DMA((NB,)),
        pltpu.SemaphoreType.DMA((NB,)),
        pltpu.SemaphoreType.DMA((NB,)),
    ),
)(_sc_body)


def _split_body(x_ref, o_ref):
    o_ref[0] = x_ref[:, :DH]
    o_ref[1] = x_ref[:, DH:]


def _tc_split(x):
    blk = 1000
    return pl.pallas_call(
        _split_body,
        grid=(N // blk,),
        in_specs=[pl.BlockSpec((blk, D), lambda i: (i, 0))],
        out_specs=pl.BlockSpec((NC, blk, DH), lambda i: (0, i, 0)),
        out_shape=jax.ShapeDtypeStruct((NC, N, DH), jnp.float32),
    )(x)


def _tc_body(p_ref, degw_ref, w0_ref, w1_ref, b_ref, out_ref):
    out_ref[...] = (
        lax.dot_general(p_ref[0], w0_ref[...], (((1,), (1,)), ((), ())),
                        preferred_element_type=jnp.float32)
        + lax.dot_general(p_ref[1], w1_ref[...], (((1,), (1,)), ((), ())),
                          preferred_element_type=jnp.float32)
        + degw_ref[...] * b_ref[...]
    )


TC_BLOCK = 1000


def _tc_call(p, degw, W0, W1, b2):
    return pl.pallas_call(
        _tc_body,
        grid=(N // TC_BLOCK,),
        in_specs=[
            pl.BlockSpec((NC, TC_BLOCK, DH), lambda i: (0, i, 0)),
            pl.BlockSpec((TC_BLOCK, 1), lambda i: (i, 0)),
            pl.BlockSpec((D, DH), lambda i: (0, 0)),
            pl.BlockSpec((D, DH), lambda i: (0, 0)),
            pl.BlockSpec((1, D), lambda i: (0, 0)),
        ],
        out_specs=pl.BlockSpec((TC_BLOCK, D), lambda i: (i, 0)),
        out_shape=jax.ShapeDtypeStruct((N, D), jnp.float32),
    )(p, degw, W0, W1, b2)


@jax.jit
def kernel(x, edge_index, edge_weight, W_lin, b_lin):
    npad = E_PAD - E
    pad_idx = (jnp.arange(npad, dtype=jnp.int32) * 13) % N
    src = jnp.concatenate([edge_index[1], pad_idx])
    dst = jnp.concatenate([edge_index[0], pad_idx])
    w = jnp.concatenate([edge_weight, jnp.zeros((npad,), jnp.float32)])
    src2 = jnp.concatenate([src, src + N]).reshape(NC * E_PAD // CHUNK, CHUNK)
    dst2 = dst.reshape(E_PAD // CHUNK, CHUNK)
    w2 = w.reshape(E_PAD // CHUNK, CHUNK)

    xs = _tc_split(x).reshape(NC * N, DH)
    agg_flat, degw_flat = _sc_call(xs, src2, dst2, w2)
    p = agg_flat.reshape(NC, N, DH)
    degw = degw_flat[:N].reshape(N, 1)
    return _tc_call(p, degw, W_lin[:, :DH], W_lin[:, DH:],
                    b_lin.reshape(1, D))


# speedup vs baseline: 7.6296x; 1.2793x over previous
import functools

import jax
import jax.numpy as jnp
from jax import lax
from jax.experimental import pallas as pl
from jax.experimental.pallas import tpu as pltpu
from jax.experimental.pallas import tpu_sc as plsc

N = 10000
E = 320000
D = 128
DH = D // 2

NC = 2
NS = 16
CHUNK = 128
E_PAD = 2560 * CHUNK
CH_PER_TILE = E_PAD // NS // CHUNK
SB = 16
NSB = CH_PER_TILE // SB
NB = 4
UNROLL = 8

ROWS_PER_TILE = 624
ROWS_TAIL = N - NS * ROWS_PER_TILE
DEGW_PAD = 10240
DEGW_PER_TILE = DEGW_PAD // NS


def _sc_body(xs_hbm, src_hbm, dst_hbm, w_hbm, agg_hbm, degw_hbm,
             agg_sh, degw_sh, src_sb, dst_sb, w_sb, dz,
             rows0, rows1, rows2, rows3, gsem, ssem, dsem):
    c = lax.axis_index("c")
    s = lax.axis_index("s")
    rows = (rows0, rows1, rows2, rows3)

    zeros16 = jnp.zeros((16,), jnp.float32)

    def _zero_rows(r, _):
        for g in range(DH // 16):
            rows0[r, pl.ds(g * 16, 16)] = zeros16
        return 0

    lax.fori_loop(0, CHUNK, _zero_rows, 0)

    def _zero_dz(i, _):
        dz[pl.ds(i * 16, 16)] = zeros16
        return 0

    lax.fori_loop(0, DEGW_PER_TILE // 16, _zero_dz, 0)

    row0 = s * ROWS_PER_TILE
    for k in range(4):
        pltpu.sync_copy(rows0.at[pl.ds(0, CHUNK)],
                        agg_sh.at[pl.ds(row0 + k * CHUNK, CHUNK)])
    pltpu.sync_copy(rows0.at[pl.ds(0, ROWS_PER_TILE - 4 * CHUNK)],
                    agg_sh.at[pl.ds(row0 + 4 * CHUNK,
                                    ROWS_PER_TILE - 4 * CHUNK)])
    pltpu.sync_copy(dz, degw_sh.at[pl.ds(s * DEGW_PER_TILE, DEGW_PER_TILE)])

    @pl.when(s == NS - 1)
    def _zero_tail():
        pltpu.sync_copy(rows0.at[pl.ds(0, ROWS_TAIL)],
                        agg_sh.at[pl.ds(NS * ROWS_PER_TILE, ROWS_TAIL)])

    plsc.subcore_barrier()

    src_row0 = c * (E_PAD // CHUNK) + s * CH_PER_TILE
    ed_row0 = s * CH_PER_TILE

    def _scale_chunk(q, buf):
        qv = jnp.full((16,), q, jnp.int32)

        @plsc.parallel_loop(0, CHUNK, 1, unroll=UNROLL)
        def _scale(e):
            ev = jnp.full((16,), e, jnp.int32)
            wb = plsc.load_gather(w_sb, [qv, ev])
            for g in range(DH // 16):
                sl = pl.ds(g * 16, 16)
                buf[e, sl] = buf[e, sl] * wb

    def _superblock(sb, _):
        pltpu.sync_copy(src_hbm.at[pl.ds(src_row0 + sb * SB, SB)], src_sb)
        pltpu.sync_copy(dst_hbm.at[pl.ds(ed_row0 + sb * SB, SB)], dst_sb)
        pltpu.sync_copy(w_hbm.at[pl.ds(ed_row0 + sb * SB, SB)], w_sb)

        def _group(qq, _):
            q0 = qq * NB
            gd = []
            for b in range(NB):
                gd.append(pltpu.async_copy(
                    xs_hbm.at[src_sb.at[q0 + b]], rows[b], gsem.at[b]))
            sd = []
            for b in range(NB):
                q = q0 + b
                gd[b].wait()
                _scale_chunk(q, rows[b])
                sd.append(pltpu.async_copy(
                    rows[b], agg_sh.at[dst_sb.at[q]], ssem.at[b], add=True))
                sd.append(pltpu.async_copy(
                    w_sb.at[q], degw_sh.at[dst_sb.at[q]], dsem.at[b],
                    add=True))
            for d in sd:
                d.wait()
            return 0

        lax.fori_loop(0, SB // NB, _group, 0)
        return 0

    lax.fori_loop(0, NSB, _superblock, 0)
    plsc.subcore_barrier()

    out_row0 = c * N + s * ROWS_PER_TILE
    pltpu.sync_copy(agg_sh.at[pl.ds(s * ROWS_PER_TILE, ROWS_PER_TILE)],
                    agg_hbm.at[pl.ds(out_row0, ROWS_PER_TILE)])

    @pl.when(s == NS - 1)
    def _drain_tail():
        pltpu.sync_copy(agg_sh.at[pl.ds(NS * ROWS_PER_TILE, ROWS_TAIL)],
                        agg_hbm.at[pl.ds(c * N + NS * ROWS_PER_TILE,
                                         ROWS_TAIL)])

    @pl.when(c == 0)
    def _drain_degw():
        pltpu.sync_copy(degw_sh.at[pl.ds(s * DEGW_PER_TILE, DEGW_PER_TILE)],
                        degw_hbm.at[pl.ds(s * DEGW_PER_TILE, DEGW_PER_TILE)])


_sc_call = functools.partial(
    pl.kernel,
    out_type=(jax.ShapeDtypeStruct((NC * N, DH), jnp.float32),
              jax.ShapeDtypeStruct((DEGW_PAD,), jnp.float32)),
    mesh=plsc.VectorSubcoreMesh(core_axis_name="c", subcore_axis_name="s"),
    compiler_params=pltpu.CompilerParams(needs_layout_passes=False, use_tc_tiling_on_sc=False),
    scratch_types=(
        pltpu.VMEM_SHARED((N, DH), jnp.float32),
        pltpu.VMEM_SHARED((DEGW_PAD,), jnp.float32),
        pltpu.VMEM((SB, CHUNK), jnp.int32),
        pltpu.VMEM((SB, CHUNK), jnp.int32),
        pltpu.VMEM((SB, CHUNK), jnp.float32),
        pltpu.VMEM((DEGW_PER_TILE,), jnp.float32),
        pltpu.VMEM((CHUNK, DH), jnp.float32),
        pltpu.VMEM((CHUNK, DH), jnp.float32),
        pltpu.VMEM((CHUNK, DH), jnp.float32),
        pltpu.VMEM((CHUNK, DH), jnp.float32),
        pltpu.SemaphoreType.DMA((NB,)),
        pltpu.SemaphoreType.DMA((NB,)),
        pltpu.SemaphoreType.DMA((NB,)),
    ),
)(_sc_body)


def _split_body(x_ref, o_ref):
    o_ref[0] = x_ref[:, :DH]
    o_ref[1] = x_ref[:, DH:]


def _tc_split(x):
    blk = 1000
    return pl.pallas_call(
        _split_body,
        grid=(N // blk,),
        in_specs=[pl.BlockSpec((blk, D), lambda i: (i, 0))],
        out_specs=pl.BlockSpec((NC, blk, DH), lambda i: (0, i, 0)),
        out_shape=jax.ShapeDtypeStruct((NC, N, DH), jnp.float32),
    )(x)


def _tc_body(p_ref, degw_ref, w0_ref, w1_ref, b_ref, out_ref):
    out_ref[...] = (
        lax.dot_general(p_ref[0], w0_ref[...], (((1,), (1,)), ((), ())),
                        preferred_element_type=jnp.float32)
        + lax.dot_general(p_ref[1], w1_ref[...], (((1,), (1,)), ((), ())),
                          preferred_element_type=jnp.float32)
        + degw_ref[...] * b_ref[...]
    )


TC_BLOCK = 1000


def _tc_call(p, degw, W0, W1, b2):
    return pl.pallas_call(
        _tc_body,
        grid=(N // TC_BLOCK,),
        in_specs=[
            pl.BlockSpec((NC, TC_BLOCK, DH), lambda i: (0, i, 0)),
            pl.BlockSpec((TC_BLOCK, 1), lambda i: (i, 0)),
            pl.BlockSpec((D, DH), lambda i: (0, 0)),
            pl.BlockSpec((D, DH), lambda i: (0, 0)),
            pl.BlockSpec((1, D), lambda i: (0, 0)),
        ],
        out_specs=pl.BlockSpec((TC_BLOCK, D), lambda i: (i, 0)),
        out_shape=jax.ShapeDtypeStruct((N, D), jnp.float32),
    )(p, degw, W0, W1, b2)


@jax.jit
def kernel(x, edge_index, edge_weight, W_lin, b_lin):
    npad = E_PAD - E
    pad_idx = (jnp.arange(npad, dtype=jnp.int32) * 13) % N
    src = jnp.concatenate([edge_index[1], pad_idx])
    dst = jnp.concatenate([edge_index[0], pad_idx])
    w = jnp.concatenate([edge_weight, jnp.zeros((npad,), jnp.float32)])
    src2 = jnp.concatenate([src, src + N]).reshape(NC * E_PAD // CHUNK, CHUNK)
    dst2 = dst.reshape(E_PAD // CHUNK, CHUNK)
    w2 = w.reshape(E_PAD // CHUNK, CHUNK)

    xs = _tc_split(x).reshape(NC * N, DH)
    agg_flat, degw_flat = _sc_call(xs, src2, dst2, w2)
    p = agg_flat.reshape(NC, N, DH)
    degw = degw_flat[:N].reshape(N, 1)
    return _tc_call(p, degw, W_lin[:, :DH], W_lin[:, DH:],
                    b_lin.reshape(1, D))


# speedup vs baseline: 7.6326x; 1.0004x over previous
import functools

import jax
import jax.numpy as jnp
from jax import lax
from jax.experimental import pallas as pl
from jax.experimental.pallas import tpu as pltpu
from jax.experimental.pallas import tpu_sc as plsc

N = 10000
E = 320000
D = 128
DH = D // 2

NC = 2
NS = 16
CHUNK = 128
E_PAD = 2560 * CHUNK
CH_PER_TILE = E_PAD // NS // CHUNK
SB = 16
NSB = CH_PER_TILE // SB
NB = 4
UNROLL = 2

ROWS_PER_TILE = 624
ROWS_TAIL = N - NS * ROWS_PER_TILE
DEGW_PAD = 10240
DEGW_PER_TILE = DEGW_PAD // NS


def _sc_body(xs_hbm, src_hbm, dst_hbm, w_hbm, agg_hbm, degw_hbm,
             agg_sh, degw_sh, src_sb, dst_sb, w_sb, dz,
             rows0, rows1, rows2, rows3, gsem, ssem, dsem):
    c = lax.axis_index("c")
    s = lax.axis_index("s")
    rows = (rows0, rows1, rows2, rows3)

    zeros16 = jnp.zeros((16,), jnp.float32)

    def _zero_rows(r, _):
        for g in range(DH // 16):
            rows0[r, pl.ds(g * 16, 16)] = zeros16
        return 0

    lax.fori_loop(0, CHUNK, _zero_rows, 0)

    def _zero_dz(i, _):
        dz[pl.ds(i * 16, 16)] = zeros16
        return 0

    lax.fori_loop(0, DEGW_PER_TILE // 16, _zero_dz, 0)

    row0 = s * ROWS_PER_TILE
    for k in range(4):
        pltpu.sync_copy(rows0.at[pl.ds(0, CHUNK)],
                        agg_sh.at[pl.ds(row0 + k * CHUNK, CHUNK)])
    pltpu.sync_copy(rows0.at[pl.ds(0, ROWS_PER_TILE - 4 * CHUNK)],
                    agg_sh.at[pl.ds(row0 + 4 * CHUNK,
                                    ROWS_PER_TILE - 4 * CHUNK)])
    pltpu.sync_copy(dz, degw_sh.at[pl.ds(s * DEGW_PER_TILE, DEGW_PER_TILE)])

    @pl.when(s == NS - 1)
    def _zero_tail():
        pltpu.sync_copy(rows0.at[pl.ds(0, ROWS_TAIL)],
                        agg_sh.at[pl.ds(NS * ROWS_PER_TILE, ROWS_TAIL)])

    plsc.subcore_barrier()

    src_row0 = c * (E_PAD // CHUNK) + s * CH_PER_TILE
    ed_row0 = s * CH_PER_TILE

    _dn = lax.GatherDimensionNumbers(
        offset_dims=(), collapsed_slice_dims=(0,), start_index_map=(0,))

    def _scale_chunk(q, buf):
        @plsc.parallel_loop(0, CHUNK // 16, 1, unroll=UNROLL)
        def _scale(g16):
            wv = w_sb[q, pl.ds(g16 * 16, 16)]
            for e in range(16):
                ev = jnp.full((16,), e, jnp.int32)
                wb = lax.gather(wv, ev[:, None], _dn, slice_sizes=(1,),
                                mode=lax.GatherScatterMode.PROMISE_IN_BOUNDS)
                row = g16 * 16 + e
                for g in range(DH // 16):
                    sl = pl.ds(g * 16, 16)
                    buf[row, sl] = buf[row, sl] * wb

    def _superblock(sb, _):
        pltpu.sync_copy(src_hbm.at[pl.ds(src_row0 + sb * SB, SB)], src_sb)
        pltpu.sync_copy(dst_hbm.at[pl.ds(ed_row0 + sb * SB, SB)], dst_sb)
        pltpu.sync_copy(w_hbm.at[pl.ds(ed_row0 + sb * SB, SB)], w_sb)

        def _group(qq, _):
            q0 = qq * NB
            gd = []
            for b in range(NB):
                gd.append(pltpu.async_copy(
                    xs_hbm.at[src_sb.at[q0 + b]], rows[b], gsem.at[b]))
            sd = []
            for b in range(NB):
                q = q0 + b
                gd[b].wait()
                _scale_chunk(q, rows[b])
                sd.append(pltpu.async_copy(
                    rows[b], agg_sh.at[dst_sb.at[q]], ssem.at[b], add=True))
                sd.append(pltpu.async_copy(
                    w_sb.at[q], degw_sh.at[dst_sb.at[q]], dsem.at[b],
                    add=True))
            for d in sd:
                d.wait()
            return 0

        lax.fori_loop(0, SB // NB, _group, 0)
        return 0

    lax.fori_loop(0, NSB, _superblock, 0)
    plsc.subcore_barrier()

    out_row0 = c * N + s * ROWS_PER_TILE
    pltpu.sync_copy(agg_sh.at[pl.ds(s * ROWS_PER_TILE, ROWS_PER_TILE)],
                    agg_hbm.at[pl.ds(out_row0, ROWS_PER_TILE)])

    @pl.when(s == NS - 1)
    def _drain_tail():
        pltpu.sync_copy(agg_sh.at[pl.ds(NS * ROWS_PER_TILE, ROWS_TAIL)],
                        agg_hbm.at[pl.ds(c * N + NS * ROWS_PER_TILE,
                                         ROWS_TAIL)])

    @pl.when(c == 0)
    def _drain_degw():
        pltpu.sync_copy(degw_sh.at[pl.ds(s * DEGW_PER_TILE, DEGW_PER_TILE)],
                        degw_hbm.at[pl.ds(s * DEGW_PER_TILE, DEGW_PER_TILE)])


_sc_call = functools.partial(
    pl.kernel,
    out_type=(jax.ShapeDtypeStruct((NC * N, DH), jnp.float32),
              jax.ShapeDtypeStruct((DEGW_PAD,), jnp.float32)),
    mesh=plsc.VectorSubcoreMesh(core_axis_name="c", subcore_axis_name="s"),
    compiler_params=pltpu.CompilerParams(needs_layout_passes=False, use_tc_tiling_on_sc=False),
    scratch_types=(
        pltpu.VMEM_SHARED((N, DH), jnp.float32),
        pltpu.VMEM_SHARED((DEGW_PAD,), jnp.float32),
        pltpu.VMEM((SB, CHUNK), jnp.int32),
        pltpu.VMEM((SB, CHUNK), jnp.int32),
        pltpu.VMEM((SB, CHUNK), jnp.float32),
        pltpu.VMEM((DEGW_PER_TILE,), jnp.float32),
        pltpu.VMEM((CHUNK, DH), jnp.float32),
        pltpu.VMEM((CHUNK, DH), jnp.float32),
        pltpu.VMEM((CHUNK, DH), jnp.float32),
        pltpu.VMEM((CHUNK, DH), jnp.float32),
        pltpu.SemaphoreType.DMA((NB,)),
        pltpu.SemaphoreType.DMA((NB,)),
        pltpu.SemaphoreType.DMA((NB,)),
    ),
)(_sc_body)


def _split_body(x_ref, o_ref):
    o_ref[0] = x_ref[:, :DH]
    o_ref[1] = x_ref[:, DH:]


def _tc_split(x):
    blk = 1000
    return pl.pallas_call(
        _split_body,
        grid=(N // blk,),
        in_specs=[pl.BlockSpec((blk, D), lambda i: (i, 0))],
        out_specs=pl.BlockSpec((NC, blk, DH), lambda i: (0, i, 0)),
        out_shape=jax.ShapeDtypeStruct((NC, N, DH), jnp.float32),
    )(x)


def _tc_body(p_ref, degw_ref, w0_ref, w1_ref, b_ref, out_ref):
    out_ref[...] = (
        lax.dot_general(p_ref[0], w0_ref[...], (((1,), (1,)), ((), ())),
                        preferred_element_type=jnp.float32)
        + lax.dot_general(p_ref[1], w1_ref[...], (((1,), (1,)), ((), ())),
                          preferred_element_type=jnp.float32)
        + degw_ref[...] * b_ref[...]
    )


TC_BLOCK = 1000


def _tc_call(p, degw, W0, W1, b2):
    return pl.pallas_call(
        _tc_body,
        grid=(N // TC_BLOCK,),
        in_specs=[
            pl.BlockSpec((NC, TC_BLOCK, DH), lambda i: (0, i, 0)),
            pl.BlockSpec((TC_BLOCK, 1), lambda i: (i, 0)),
            pl.BlockSpec((D, DH), lambda i: (0, 0)),
            pl.BlockSpec((D, DH), lambda i: (0, 0)),
            pl.BlockSpec((1, D), lambda i: (0, 0)),
        ],
        out_specs=pl.BlockSpec((TC_BLOCK, D), lambda i: (i, 0)),
        out_shape=jax.ShapeDtypeStruct((N, D), jnp.float32),
    )(p, degw, W0, W1, b2)


@jax.jit
def kernel(x, edge_index, edge_weight, W_lin, b_lin):
    npad = E_PAD - E
    pad_idx = (jnp.arange(npad, dtype=jnp.int32) * 13) % N
    src = jnp.concatenate([edge_index[1], pad_idx])
    dst = jnp.concatenate([edge_index[0], pad_idx])
    w = jnp.concatenate([edge_weight, jnp.zeros((npad,), jnp.float32)])
    src2 = jnp.concatenate([src, src + N]).reshape(NC * E_PAD // CHUNK, CHUNK)
    dst2 = dst.reshape(E_PAD // CHUNK, CHUNK)
    w2 = w.reshape(E_PAD // CHUNK, CHUNK)

    xs = _tc_split(x).reshape(NC * N, DH)
    agg_flat, degw_flat = _sc_call(xs, src2, dst2, w2)
    p = agg_flat.reshape(NC, N, DH)
    degw = degw_flat[:N].reshape(N, 1)
    return _tc_call(p, degw, W_lin[:, :DH], W_lin[:, DH:],
                    b_lin.reshape(1, D))


# speedup vs baseline: 9.3173x; 1.2207x over previous
import functools

import jax
import jax.numpy as jnp
from jax import lax
from jax.experimental import pallas as pl
from jax.experimental.pallas import tpu as pltpu
from jax.experimental.pallas import tpu_sc as plsc

N = 10000
E = 320000
D = 128
DH = D // 2

NC = 2
NS = 16
CHUNK = 128
E_PAD = 2560 * CHUNK
CH_PER_TILE = E_PAD // NS // CHUNK
SB = 16
NSB = CH_PER_TILE // SB
NB = 8
UNROLL = 2

ROWS_PER_TILE = 624
ROWS_TAIL = N - NS * ROWS_PER_TILE


def _sc_body(sup_hbm, src_hbm, dst_hbm, w_hbm, out_hbm,
             agg_sh, src_sb, dst_sb, w_sb,
             rows0, rows1, rows2, rows3, rows4, rows5, rows6, rows7,
             gsem, ssem):
    c = lax.axis_index("c")
    s = lax.axis_index("s")
    rows = (rows0, rows1, rows2, rows3, rows4, rows5, rows6, rows7)

    zeros16 = jnp.zeros((16,), jnp.float32)

    def _zero_rows(r, _):
        for g in range(DH // 16):
            rows0[r, pl.ds(g * 16, 16)] = zeros16
        return 0

    lax.fori_loop(0, CHUNK, _zero_rows, 0)

    row0 = s * ROWS_PER_TILE
    for k in range(4):
        pltpu.sync_copy(rows0.at[pl.ds(0, CHUNK)],
                        agg_sh.at[pl.ds(row0 + k * CHUNK, CHUNK)])
    pltpu.sync_copy(rows0.at[pl.ds(0, ROWS_PER_TILE - 4 * CHUNK)],
                    agg_sh.at[pl.ds(row0 + 4 * CHUNK,
                                    ROWS_PER_TILE - 4 * CHUNK)])

    @pl.when(s == NS - 1)
    def _zero_tail():
        pltpu.sync_copy(rows0.at[pl.ds(0, ROWS_TAIL)],
                        agg_sh.at[pl.ds(NS * ROWS_PER_TILE, ROWS_TAIL)])

    plsc.subcore_barrier()

    src_row0 = c * (E_PAD // CHUNK) + s * CH_PER_TILE
    ed_row0 = s * CH_PER_TILE

    _dn = lax.GatherDimensionNumbers(
        offset_dims=(), collapsed_slice_dims=(0,), start_index_map=(0,))

    def _scale_chunk(q, buf):
        @plsc.parallel_loop(0, CHUNK // 16, 1, unroll=UNROLL)
        def _scale(g16):
            wv = w_sb[q, pl.ds(g16 * 16, 16)]
            for e in range(16):
                ev = jnp.full((16,), e, jnp.int32)
                wb = lax.gather(wv, ev[:, None], _dn, slice_sizes=(1,),
                                mode=lax.GatherScatterMode.PROMISE_IN_BOUNDS)
                row = g16 * 16 + e
                for g in range(DH // 16):
                    sl = pl.ds(g * 16, 16)
                    buf[row, sl] = buf[row, sl] * wb

    def _superblock(sb, _):
        pltpu.sync_copy(src_hbm.at[pl.ds(src_row0 + sb * SB, SB)], src_sb)
        pltpu.sync_copy(dst_hbm.at[pl.ds(ed_row0 + sb * SB, SB)], dst_sb)
        pltpu.sync_copy(w_hbm.at[pl.ds(ed_row0 + sb * SB, SB)], w_sb)

        def _group(qq, _):
            q0 = qq * NB
            gd = []
            for b in range(NB):
                gd.append(pltpu.async_copy(
                    sup_hbm.at[src_sb.at[q0 + b]], rows[b], gsem.at[b]))
            sd = []
            for b in range(NB):
                q = q0 + b
                gd[b].wait()
                _scale_chunk(q, rows[b])
                sd.append(pltpu.async_copy(
                    rows[b], agg_sh.at[dst_sb.at[q]], ssem.at[b], add=True))
            for d in sd:
                d.wait()
            return 0

        lax.fori_loop(0, SB // NB, _group, 0)
        return 0

    lax.fori_loop(0, NSB, _superblock, 0)
    plsc.subcore_barrier()

    pltpu.sync_copy(agg_sh.at[pl.ds(s * ROWS_PER_TILE, ROWS_PER_TILE)],
                    out_hbm.at[pl.ds(s * ROWS_PER_TILE, ROWS_PER_TILE),
                               pl.ds(c * DH, DH)])

    @pl.when(s == NS - 1)
    def _drain_tail():
        pltpu.sync_copy(agg_sh.at[pl.ds(NS * ROWS_PER_TILE, ROWS_TAIL)],
                        out_hbm.at[pl.ds(NS * ROWS_PER_TILE, ROWS_TAIL),
                                   pl.ds(c * DH, DH)])


_sc_call = functools.partial(
    pl.kernel,
    out_type=jax.ShapeDtypeStruct((N, D), jnp.float32),
    mesh=plsc.VectorSubcoreMesh(core_axis_name="c", subcore_axis_name="s"),
    compiler_params=pltpu.CompilerParams(needs_layout_passes=False,
                                         use_tc_tiling_on_sc=False),
    scratch_types=(
        pltpu.VMEM_SHARED((N, DH), jnp.float32),
        pltpu.VMEM((SB, CHUNK), jnp.int32),
        pltpu.VMEM((SB, CHUNK), jnp.int32),
        pltpu.VMEM((SB, CHUNK), jnp.float32),
        pltpu.VMEM((CHUNK, DH), jnp.float32),
        pltpu.VMEM((CHUNK, DH), jnp.float32),
        pltpu.VMEM((CHUNK, DH), jnp.float32),
        pltpu.VMEM((CHUNK, DH), jnp.float32),
        pltpu.VMEM((CHUNK, DH), jnp.float32),
        pltpu.VMEM((CHUNK, DH), jnp.float32),
        pltpu.VMEM((CHUNK, DH), jnp.float32),
        pltpu.VMEM((CHUNK, DH), jnp.float32),
        pltpu.SemaphoreType.DMA((NB,)),
        pltpu.SemaphoreType.DMA((NB,)),
    ),
)(_sc_body)


TC_BLOCK = 1000


def _mm_body(x_ref, w_ref, b_ref, o_ref):
    res = (lax.dot_general(x_ref[...], w_ref[...], (((1,), (1,)), ((), ())),
                           preferred_element_type=jnp.float32)
           + b_ref[...])
    o_ref[0] = res[:, :DH]
    o_ref[1] = res[:, DH:]


def _tc_matmul(x, W_lin, b2):
    return pl.pallas_call(
        _mm_body,
        grid=(N // TC_BLOCK,),
        in_specs=[
            pl.BlockSpec((TC_BLOCK, D), lambda i: (i, 0)),
            pl.BlockSpec((D, D), lambda i: (0, 0)),
            pl.BlockSpec((1, D), lambda i: (0, 0)),
        ],
        out_specs=pl.BlockSpec((NC, TC_BLOCK, DH), lambda i: (0, i, 0)),
        out_shape=jax.ShapeDtypeStruct((NC, N, DH), jnp.float32),
    )(x, W_lin, b2)


@jax.jit
def kernel(x, edge_index, edge_weight, W_lin, b_lin):
    npad = E_PAD - E
    pad_idx = (jnp.arange(npad, dtype=jnp.int32) * 13) % N
    src = jnp.concatenate([edge_index[1], pad_idx])
    dst = jnp.concatenate([edge_index[0], pad_idx])
    w = jnp.concatenate([edge_weight, jnp.zeros((npad,), jnp.float32)])
    src2 = jnp.concatenate([src, src + N]).reshape(NC * E_PAD // CHUNK, CHUNK)
    dst2 = dst.reshape(E_PAD // CHUNK, CHUNK)
    w2 = w.reshape(E_PAD // CHUNK, CHUNK)

    sup = _tc_matmul(x, W_lin, b_lin.reshape(1, D)).reshape(NC * N, DH)
    return _sc_call(sup, src2, dst2, w2)


# speedup vs baseline: 9.6876x; 1.0397x over previous
import functools

import jax
import jax.numpy as jnp
from jax import lax
from jax.experimental import pallas as pl
from jax.experimental.pallas import tpu as pltpu
from jax.experimental.pallas import tpu_sc as plsc

N = 10000
E = 320000
D = 128
DH = D // 2

NC = 2
NS = 16
CHUNK = 128
E_PAD = 2560 * CHUNK
CH_PER_TILE = E_PAD // NS // CHUNK
SB = 16
NSB = CH_PER_TILE // SB
NB = 8
UNROLL = 2

ROWS_PER_TILE = 624
ROWS_TAIL = N - NS * ROWS_PER_TILE


def _sc_body(sup_hbm, src_hbm, dst_hbm, w_hbm, out_hbm,
             agg_sh, src_sb, dst_sb, w_sb,
             rows0, rows1, rows2, rows3, rows4, rows5, rows6, rows7,
             gsem, ssem):
    c = lax.axis_index("c")
    s = lax.axis_index("s")
    rows = (rows0, rows1, rows2, rows3, rows4, rows5, rows6, rows7)

    zeros16 = jnp.zeros((16,), jnp.float32)

    def _zero_rows(r, _):
        for g in range(DH // 16):
            rows0[r, pl.ds(g * 16, 16)] = zeros16
        return 0

    lax.fori_loop(0, CHUNK, _zero_rows, 0)

    row0 = s * ROWS_PER_TILE
    for k in range(4):
        pltpu.sync_copy(rows0.at[pl.ds(0, CHUNK)],
                        agg_sh.at[pl.ds(row0 + k * CHUNK, CHUNK)])
    pltpu.sync_copy(rows0.at[pl.ds(0, ROWS_PER_TILE - 4 * CHUNK)],
                    agg_sh.at[pl.ds(row0 + 4 * CHUNK,
                                    ROWS_PER_TILE - 4 * CHUNK)])

    @pl.when(s == NS - 1)
    def _zero_tail():
        pltpu.sync_copy(rows0.at[pl.ds(0, ROWS_TAIL)],
                        agg_sh.at[pl.ds(NS * ROWS_PER_TILE, ROWS_TAIL)])

    plsc.subcore_barrier()

    ed_row0 = s * CH_PER_TILE
    cn = jnp.full((16,), 1, jnp.int32) * (c * N)

    _dn = lax.GatherDimensionNumbers(
        offset_dims=(), collapsed_slice_dims=(0,), start_index_map=(0,))

    def _scale_chunk(q, buf):
        @plsc.parallel_loop(0, CHUNK // 16, 1, unroll=UNROLL)
        def _scale(g16):
            wv = w_sb[q, pl.ds(g16 * 16, 16)]
            for e in range(16):
                ev = jnp.full((16,), e, jnp.int32)
                wb = lax.gather(wv, ev[:, None], _dn, slice_sizes=(1,),
                                mode=lax.GatherScatterMode.PROMISE_IN_BOUNDS)
                row = g16 * 16 + e
                for g in range(DH // 16):
                    sl = pl.ds(g * 16, 16)
                    buf[row, sl] = buf[row, sl] * wb

    def _superblock(sb, _):
        pltpu.sync_copy(src_hbm.at[pl.ds(ed_row0 + sb * SB, SB)], src_sb)
        pltpu.sync_copy(dst_hbm.at[pl.ds(ed_row0 + sb * SB, SB)], dst_sb)
        pltpu.sync_copy(w_hbm.at[pl.ds(ed_row0 + sb * SB, SB)], w_sb)

        def _adj(r, _):
            for g in range(CHUNK // 16):
                sl = pl.ds(g * 16, 16)
                src_sb[r, sl] = src_sb[r, sl] + cn
            return 0

        lax.fori_loop(0, SB, _adj, 0)

        def _group(qq, _):
            q0 = qq * NB
            gd = []
            for b in range(NB):
                @pl.when(qq > 0)
                def _wait_prev(b=b):
                    pltpu.make_async_copy(
                        rows[b], agg_sh.at[dst_sb.at[0]], ssem.at[b]).wait()
                gd.append(pltpu.async_copy(
                    sup_hbm.at[src_sb.at[q0 + b]], rows[b], gsem.at[b]))
            for b in range(NB):
                q = q0 + b
                gd[b].wait()
                _scale_chunk(q, rows[b])
                pltpu.async_copy(
                    rows[b], agg_sh.at[dst_sb.at[q]], ssem.at[b], add=True)
            return 0

        lax.fori_loop(0, SB // NB, _group, 0)
        for b in range(NB):
            pltpu.make_async_copy(
                rows[b], agg_sh.at[dst_sb.at[0]], ssem.at[b]).wait()
        return 0

    lax.fori_loop(0, NSB, _superblock, 0)
    plsc.subcore_barrier()

    pltpu.sync_copy(agg_sh.at[pl.ds(s * ROWS_PER_TILE, ROWS_PER_TILE)],
                    out_hbm.at[pl.ds(s * ROWS_PER_TILE, ROWS_PER_TILE),
                               pl.ds(c * DH, DH)])

    @pl.when(s == NS - 1)
    def _drain_tail():
        pltpu.sync_copy(agg_sh.at[pl.ds(NS * ROWS_PER_TILE, ROWS_TAIL)],
                        out_hbm.at[pl.ds(NS * ROWS_PER_TILE, ROWS_TAIL),
                                   pl.ds(c * DH, DH)])


_sc_call = functools.partial(
    pl.kernel,
    out_type=jax.ShapeDtypeStruct((N, D), jnp.float32),
    mesh=plsc.VectorSubcoreMesh(core_axis_name="c", subcore_axis_name="s"),
    compiler_params=pltpu.CompilerParams(needs_layout_passes=False,
                                         use_tc_tiling_on_sc=False),
    scratch_types=(
        pltpu.VMEM_SHARED((N, DH), jnp.float32),
        pltpu.VMEM((SB, CHUNK), jnp.int32),
        pltpu.VMEM((SB, CHUNK), jnp.int32),
        pltpu.VMEM((SB, CHUNK), jnp.float32),
        pltpu.VMEM((CHUNK, DH), jnp.float32),
        pltpu.VMEM((CHUNK, DH), jnp.float32),
        pltpu.VMEM((CHUNK, DH), jnp.float32),
        pltpu.VMEM((CHUNK, DH), jnp.float32),
        pltpu.VMEM((CHUNK, DH), jnp.float32),
        pltpu.VMEM((CHUNK, DH), jnp.float32),
        pltpu.VMEM((CHUNK, DH), jnp.float32),
        pltpu.VMEM((CHUNK, DH), jnp.float32),
        pltpu.SemaphoreType.DMA((NB,)),
        pltpu.SemaphoreType.DMA((NB,)),
    ),
)(_sc_body)


TC_BLOCK = 1000


def _mm_body(x_ref, w_ref, b_ref, o_ref):
    res = (lax.dot_general(x_ref[...], w_ref[...], (((1,), (1,)), ((), ())),
                           preferred_element_type=jnp.float32)
           + b_ref[...])
    o_ref[0] = res[:, :DH]
    o_ref[1] = res[:, DH:]


def _tc_matmul(x, W_lin, b2):
    return pl.pallas_call(
        _mm_body,
        grid=(N // TC_BLOCK,),
        in_specs=[
            pl.BlockSpec((TC_BLOCK, D), lambda i: (i, 0)),
            pl.BlockSpec((D, D), lambda i: (0, 0)),
            pl.BlockSpec((1, D), lambda i: (0, 0)),
        ],
        out_specs=pl.BlockSpec((NC, TC_BLOCK, DH), lambda i: (0, i, 0)),
        out_shape=jax.ShapeDtypeStruct((NC, N, DH), jnp.float32),
    )(x, W_lin, b2)


@jax.jit
def kernel(x, edge_index, edge_weight, W_lin, b_lin):
    npad = E_PAD - E
    pad_idx = (jnp.arange(npad, dtype=jnp.int32) * 13) % N
    src = jnp.concatenate([edge_index[1], pad_idx])
    dst = jnp.concatenate([edge_index[0], pad_idx])
    w = jnp.concatenate([edge_weight, jnp.zeros((npad,), jnp.float32)])
    src2 = src.reshape(E_PAD // CHUNK, CHUNK)
    dst2 = dst.reshape(E_PAD // CHUNK, CHUNK)
    w2 = w.reshape(E_PAD // CHUNK, CHUNK)

    sup = _tc_matmul(x, W_lin, b_lin.reshape(1, D)).reshape(NC * N, DH)
    return _sc_call(sup, src2, dst2, w2)
